# Initial kernel scaffold; baseline (speedup 1.0000x reference)
import jax
import jax.numpy as jnp
from jax.experimental import pallas as pl


def kernel(x, conv_w, conv_b, codebook):
    raise NotImplementedError("write your pallas kernel here")


# speedup vs baseline: 1.9378x; 1.9378x over previous
import jax
import jax.numpy as jnp
from jax import lax
from jax.experimental import pallas as pl
from jax.experimental.pallas import tpu as pltpu

_B, _C, _H, _W = 4, 192, 16, 16
_HW = _H * _W
_K, _D = 1024, 64


def _vq_body(x_ref, w_ref, b_ref, cb_ref, enc_ref, idx_ref, emb_ref, out_ref):
    xb = x_ref[0]
    w = w_ref[...]
    enc = jnp.dot(w, xb, preferred_element_type=jnp.float32,
                  precision=lax.Precision.DEFAULT) + b_ref[...]
    cb = cb_ref[...]
    scores = jnp.dot(cb, enc, preferred_element_type=jnp.float32,
                     precision=lax.Precision.HIGHEST)
    cnorm = jnp.sum(cb * cb, axis=1, keepdims=True)
    dist = cnorm - 2.0 * scores
    minv = jnp.min(dist, axis=0, keepdims=True)
    kiota = lax.broadcasted_iota(jnp.int32, (_K, _HW), 0)
    idx = jnp.min(jnp.where(dist == minv, kiota, _K),
                  axis=0, keepdims=True)
    idx_ref[0] = idx
    onehot = (kiota == idx).astype(jnp.float32)
    emb = lax.dot_general(cb, onehot, (((0,), (0,)), ((), ())),
                          preferred_element_type=jnp.float32,
                          precision=lax.Precision.HIGHEST)
    enc_ref[0] = enc
    emb_ref[0] = emb
    out_ref[0] = enc + (emb - enc)


def kernel(x, conv_w, conv_b, codebook):
    xr = x.reshape(_B, _C, _HW)
    b2 = conv_b.reshape(_D, 1)
    grid = (_B,)
    enc, idx, emb, out = pl.pallas_call(
        _vq_body,
        grid=grid,
        in_specs=[
            pl.BlockSpec((1, _C, _HW), lambda b: (b, 0, 0)),
            pl.BlockSpec((_D, _C), lambda b: (0, 0)),
            pl.BlockSpec((_D, 1), lambda b: (0, 0)),
            pl.BlockSpec((_K, _D), lambda b: (0, 0)),
        ],
        out_specs=[
            pl.BlockSpec((1, _D, _HW), lambda b: (b, 0, 0)),
            pl.BlockSpec((1, 1, _HW), lambda b: (b, 0, 0)),
            pl.BlockSpec((1, _D, _HW), lambda b: (b, 0, 0)),
            pl.BlockSpec((1, _D, _HW), lambda b: (b, 0, 0)),
        ],
        out_shape=[
            jax.ShapeDtypeStruct((_B, _D, _HW), jnp.float32),
            jax.ShapeDtypeStruct((_B, 1, _HW), jnp.int32),
            jax.ShapeDtypeStruct((_B, _D, _HW), jnp.float32),
            jax.ShapeDtypeStruct((_B, _D, _HW), jnp.float32),
        ],
    )(xr, conv_w, b2, codebook)
    return (out.reshape(_B, _D, _H, _W),
            emb.reshape(_B, _D, _H, _W),
            enc.reshape(_B, _D, _H, _W),
            idx.reshape(_B, _H, _W))
